# Initial kernel scaffold; baseline (speedup 1.0000x reference)
import jax
import jax.numpy as jnp
from jax.experimental import pallas as pl


def kernel(nodes, senders, receivers, globals_, params):
    raise NotImplementedError("write your pallas kernel here")


# speedup vs baseline: 2.3021x; 2.3021x over previous
import functools

import jax
import jax.numpy as jnp
from jax import lax
from jax.experimental import pallas as pl
from jax.experimental.pallas import tpu as pltpu
from jax.experimental.pallas import tpu_sc as plsc

N = 50000
E = 800000
F = 16
H = 10
D = 64
STEPS = 3

NC = 2
NS = 16
NW = NC * NS

GCH = 896

PE = 16 * 49 * 1024
EPT_S = PE // NS
R_SC = 25088
TRASH = R_SC
ACC_ROWS = 25120
ZPT = ACC_ROWS // NS
OPT = R_SC // NS
NAGG = NC * R_SC

BE = 2000
BN = 2000


def _swish(x):
    return x * jax.nn.sigmoid(x)


def _bdot(a, w):
    return jnp.dot(a.astype(jnp.bfloat16), w,
                   preferred_element_type=jnp.float32)


def _sc_gather_body(x_hbm, snd_hbm, rcv_hbm, sent_out, recv_out,
                    i0, i1, i2, i3, i4, i5, i6,
                    r0, r1, r2, r3, r4, r5, r6, sem):
    c = lax.axis_index("c")
    s = lax.axis_index("s")
    wid = s * NC + c
    ept = PE // NW
    idx7 = (i0, i1, i2, i3, i4, i5, i6)
    rows7 = (r0, r1, r2, r3, r4, r5, r6)

    def chunk(i, carry):
        base = wid * ept + i * GCH
        for src_hbm, out_hbm in ((snd_hbm, sent_out), (rcv_hbm, recv_out)):
            for j in range(GCH // 128):
                pltpu.sync_copy(src_hbm.at[pl.ds(base + j * 128, 128)],
                                idx7[j])
            descs = []
            for j in range(GCH // 128):
                descs.append(pltpu.async_copy(
                    x_hbm.at[idx7[j]], rows7[j], sem))
            for d_ in descs:
                d_.wait()
            for j in range(GCH // 128):
                pltpu.sync_copy(rows7[j],
                                out_hbm.at[pl.ds(base + j * 128, 128)])
        return carry

    lax.fori_loop(0, (PE // NW) // GCH, chunk, 0)


_sc_gather = functools.partial(
    pl.kernel,
    out_type=(jax.ShapeDtypeStruct((PE, F), jnp.float32),
              jax.ShapeDtypeStruct((PE, F), jnp.float32)),
    compiler_params=pltpu.CompilerParams(use_tc_tiling_on_sc=False),
    mesh=plsc.VectorSubcoreMesh(core_axis_name="c", subcore_axis_name="s"),
    scratch_types=(
        [pltpu.VMEM((128,), jnp.int32) for _ in range(7)]
        + [pltpu.VMEM((128, F), jnp.float32) for _ in range(7)]
        + [pltpu.SemaphoreType.DMA]
    ),
)(_sc_gather_body)


def _make_sc_scatter(pd, sch):
    n_chunks = EPT_S // sch
    assert EPT_S % sch == 0 and sch % 128 == 0

    def body(pay_hbm, ridx_hbm, agg_out, pbuf, ridx_v, lidx2, acc):
        c = lax.axis_index("c")
        s = lax.axis_index("s")
        lo = c * R_SC

        def zrow(r, carry):
            for j in range(pd // 16):
                pbuf[r, pl.ds(j * 16, 16)] = jnp.zeros((16,), jnp.float32)
            return carry
        lax.fori_loop(0, sch, zrow, 0)
        zfull, zrem = ZPT // sch, ZPT % sch
        for z in range(zfull):
            pltpu.sync_copy(pbuf, acc.at[pl.ds(s * ZPT + z * sch, sch)])
        if zrem:
            pltpu.sync_copy(pbuf.at[pl.ds(0, zrem)],
                            acc.at[pl.ds(s * ZPT + zfull * sch, zrem)])
        plsc.subcore_barrier()

        def chunk(i, carry):
            base = s * EPT_S + i * sch
            pltpu.sync_copy(ridx_hbm.at[pl.ds(base, sch)], ridx_v)

            def cmp(k, cc):
                v = ridx_v[pl.ds(k * 16, 16)]
                l = v - lo
                ok = (l >= 0) & (l < R_SC)
                l = jnp.where(ok, l, TRASH)
                lidx2[k >> 3, pl.ds((k & 7) * 16, 16)] = l
                return cc
            lax.fori_loop(0, sch // 16, cmp, 0)

            pltpu.sync_copy(pay_hbm.at[pl.ds(base, sch)], pbuf)
            for j in range(sch // 128):
                pltpu.sync_copy(pbuf.at[pl.ds(j * 128, 128)],
                                acc.at[lidx2.at[j]], add=True)
            return carry

        lax.fori_loop(0, n_chunks, chunk, 0)
        plsc.subcore_barrier()
        pltpu.sync_copy(acc.at[pl.ds(s * OPT, OPT)],
                        agg_out.at[pl.ds(lo + s * OPT, OPT)])

    return functools.partial(
        pl.kernel,
        out_type=jax.ShapeDtypeStruct((NAGG, pd), jnp.float32),
        compiler_params=pltpu.CompilerParams(use_tc_tiling_on_sc=False),
        mesh=plsc.VectorSubcoreMesh(core_axis_name="c", subcore_axis_name="s"),
        scratch_types=[
            pltpu.VMEM((sch, pd), jnp.float32),
            pltpu.VMEM((sch,), jnp.int32),
            pltpu.VMEM((sch // 128, 128), jnp.int32),
            pltpu.VMEM_SHARED((ACC_ROWS, pd), jnp.float32),
        ],
    )(body)


_sc_scatter_m = _make_sc_scatter(D, 256)
_sc_scatter_ev = _make_sc_scatter(16, 1024)


def _edge_body(sent_ref, recv_ref, A, B, w20, c0, W1, b1, W2, b2,
               Wx0, bx0, Wx1, bx1, Wx2, bx2, out_m_ref, out_ev_ref):
    sv = sent_ref[...]
    rv = recv_ref[...]
    d = sv - rv
    d3 = d[:, 0:3]
    d2 = jnp.sqrt(jnp.sum(d3 * d3, axis=1, keepdims=True)) ** 2
    d2b = d2.astype(jnp.bfloat16).astype(jnp.float32)
    pre = (_bdot(sv, A[...]) + _bdot(rv, B[...])
           + d2b * w20[...].astype(jnp.float32) + c0[...])
    h = _swish(pre)
    h = _swish(_bdot(h, W1[...]) + b1[...])
    m_ij = _bdot(h, W2[...]) + b2[...]
    t = _swish(_bdot(m_ij, Wx0[...]) + bx0[...])
    t = _swish(_bdot(t, Wx1[...]) + bx1[...])
    xsc = _bdot(t, Wx2[...]) + bx2[...]
    ev = d3 * xsc
    out_m_ref[...] = m_ij
    out_ev_ref[...] = jnp.concatenate(
        [ev, jnp.zeros((m_ij.shape[0], 13), jnp.float32)], axis=-1)


def _tc_edge(sent, recv, ew):
    full = lambda shape: pl.BlockSpec(shape, lambda i: (0, 0))
    in_specs = [
        pl.BlockSpec((BE, F), lambda i: (i, 0)),
        pl.BlockSpec((BE, F), lambda i: (i, 0)),
        full((F, D)), full((F, D)), full((1, D)), full((1, D)),
        full((D, D)), full((1, D)), full((D, D)), full((1, D)),
        full((D, D)), full((1, D)), full((D, D)), full((1, D)),
        full((D, 1)), full((1, 1)),
    ]
    return pl.pallas_call(
        _edge_body,
        grid=(E // BE,),
        in_specs=in_specs,
        out_specs=(pl.BlockSpec((BE, D), lambda i: (i, 0)),
                   pl.BlockSpec((BE, 16), lambda i: (i, 0))),
        out_shape=(jax.ShapeDtypeStruct((PE, D), jnp.float32),
                   jax.ShapeDtypeStruct((PE, 16), jnp.float32)),
        compiler_params=pltpu.CompilerParams(
            dimension_semantics=("arbitrary",)),
    )(sent, recv, *ew)


def _node_body(x_ref, aggm_ref, aggev_ref, Wv0, bv0, Wv1, bv1, Wv2, bv2,
               Wh0a, Wh0b, bh0, Wh1, bh1, Wh2, bh2,
               xs_ref, vs_ref, lns_ref, lnb_ref, out_ref):
    x = x_ref[...]
    m_i = aggm_ref[...]
    sum_x = aggev_ref[...][:, 0:3]
    x_n = x[:, 0:3]
    v_n = x[:, 3:6]
    h_n = x[:, 6:F]

    t = _swish(_bdot(x, Wv0[...]) + bv0[...])
    t = _swish(_bdot(t, Wv1[...]) + bv1[...])
    vfac = _bdot(t, Wv2[...]) + bv2[...]

    v_p = sum_x * (1.0 / (E - 1)) + vfac * v_n
    x_p = x_n + v_p

    u = _swish(_bdot(x, Wh0a[...]) + _bdot(m_i, Wh0b[...]) + bh0[...])
    u = _swish(_bdot(u, Wh1[...]) + bh1[...])
    h_p = _bdot(u, Wh2[...]) + bh2[...] + h_n

    def cnorm(cv, scale):
        nrm = jnp.sqrt(jnp.sum(cv * cv, axis=1, keepdims=True))
        return cv / jnp.maximum(nrm, 1e-5) * scale

    x_p = cnorm(x_p, xs_ref[...])
    v_p = cnorm(v_p, vs_ref[...])

    mu = jnp.mean(h_p, axis=1, keepdims=True)
    var = jnp.mean((h_p - mu) * (h_p - mu), axis=1, keepdims=True)
    h_p = (h_p - mu) * lax.rsqrt(var + 1e-6) * lns_ref[...] + lnb_ref[...]

    out_ref[...] = jnp.concatenate([x_p, v_p, h_p], axis=-1)


def _tc_node(x_all, agg_m, agg_ev, nw):
    full = lambda shape: pl.BlockSpec(shape, lambda i: (0, 0))
    in_specs = [
        pl.BlockSpec((BN, F), lambda i: (i, 0)),
        pl.BlockSpec((BN, D), lambda i: (i, 0)),
        pl.BlockSpec((BN, 16), lambda i: (i, 0)),
        full((F, D)), full((1, D)), full((D, D)), full((1, D)),
        full((D, 1)), full((1, 1)),
        full((F, D)), full((D, D)), full((1, D)),
        full((D, D)), full((1, D)), full((D, H)), full((1, H)),
        full((1, 1)), full((1, 1)), full((1, H)), full((1, H)),
    ]
    return pl.pallas_call(
        _node_body,
        grid=(N // BN,),
        in_specs=in_specs,
        out_specs=pl.BlockSpec((BN, F), lambda i: (i, 0)),
        out_shape=jax.ShapeDtypeStruct((N, F), jnp.float32),
        compiler_params=pltpu.CompilerParams(
            dimension_semantics=("arbitrary",)),
    )(x_all, agg_m, agg_ev, *nw)


def _prep_step_weights(params, g, s):
    p = lambda name: params["s%d_%s" % (s, name)]
    bf = lambda a: a.astype(jnp.bfloat16)
    z = jnp.zeros((F, D), jnp.float32)
    We0 = p("e_W0")
    A = bf(z.at[6:F].set(We0[0:H]))
    B = bf(z.at[6:F].set(We0[H:2 * H]))
    w20 = bf(We0[2 * H:2 * H + 1])
    gb = bf(g).astype(jnp.float32)
    c0 = gb @ bf(We0[2 * H + 1:]).astype(jnp.float32) + p("e_b0")[None, :]
    ew = (A, B, w20, c0,
          bf(p("e_W1")), p("e_b1")[None, :], bf(p("e_W2")), p("e_b2")[None, :],
          bf(p("x_W0")), p("x_b0")[None, :], bf(p("x_W1")), p("x_b1")[None, :],
          bf(p("x_W2")), p("x_b2")[None, :])

    Wv0 = bf(jnp.zeros((F, D), jnp.float32).at[6:F].set(p("v_W0")))
    Wh0 = p("h_W0")
    Wh0a = bf(jnp.zeros((F, D), jnp.float32).at[6:F].set(Wh0[0:H]))
    Wh0b = bf(Wh0[H:])
    nw = (Wv0, p("v_b0")[None, :], bf(p("v_W1")), p("v_b1")[None, :],
          bf(p("v_W2")), p("v_b2")[None, :],
          Wh0a, Wh0b, p("h_b0")[None, :],
          bf(p("h_W1")), p("h_b1")[None, :], bf(p("h_W2")), p("h_b2")[None, :],
          p("xscale")[None, :], p("vscale")[None, :],
          p("ln_scale")[None, :], p("ln_bias")[None, :])
    return ew, nw


def kernel(nodes, senders, receivers, globals_, params):
    g = globals_.reshape(1, -1).astype(jnp.float32)
    senders_pad = jnp.concatenate(
        [senders, jnp.zeros((PE - E,), jnp.int32)])
    receivers_pad0 = jnp.concatenate(
        [receivers, jnp.zeros((PE - E,), jnp.int32)])
    receivers_pad = jnp.concatenate(
        [receivers, jnp.full((PE - E,), N, jnp.int32)])
    x_all = nodes
    for s in range(STEPS):
        ew, nw = _prep_step_weights(params, g, s)
        sent, recv = _sc_gather(x_all, senders_pad, receivers_pad0)
        pay_m, pay_ev = _tc_edge(sent, recv, ew)
        agg_m = _sc_scatter_m(pay_m, receivers_pad)
        agg_ev = _sc_scatter_ev(pay_ev, receivers_pad)
        x_all = _tc_node(x_all, agg_m, agg_ev, nw)
    return x_all


# speedup vs baseline: 2.4369x; 1.0586x over previous
import functools

import jax
import jax.numpy as jnp
from jax import lax
from jax.experimental import pallas as pl
from jax.experimental.pallas import tpu as pltpu
from jax.experimental.pallas import tpu_sc as plsc

N = 50000
E = 800000
F = 16
H = 10
D = 64
STEPS = 3

NC = 2
NS = 16
NW = NC * NS

GCH = 896

PE = 16 * 49 * 1024
EPT_S = PE // NS
SCH = 512
NCHUNK = EPT_S // SCH
CPB = 6
R_SC = 25088
TRASH = R_SC
ACC_ROWS = 25104
ZPT = ACC_ROWS // NS
ZR = 16
OPT = R_SC // NS
NAGG = NC * R_SC

BE = 2000
BN = 2000


def _swish(x):
    return x * jax.nn.sigmoid(x)


def _bdot(a, w):
    return jnp.dot(a.astype(jnp.bfloat16), w,
                   preferred_element_type=jnp.float32)


def _sc_gather_body(x_hbm, snd_hbm, rcv_hbm, sent_out, recv_out,
                    idx_a, idx_b, rows_a, rows_b, sem_g, sem_o):
    c = lax.axis_index("c")
    s = lax.axis_index("s")
    wid = s * NC + c
    ept = PE // NW
    nch = ept // GCH

    def do_unit(base, idx_v, rows_v, src_hbm):
        pltpu.sync_copy(src_hbm.at[pl.ds(base, GCH)], idx_v)
        descs = []
        for j in range(GCH // 128):
            descs.append(pltpu.async_copy(
                x_hbm.at[idx_v.at[pl.ds(j * 128, 128)]],
                rows_v.at[pl.ds(j * 128, 128)], sem_g))
        return descs

    def drain_out():
        pltpu.make_async_copy(rows_a, sent_out.at[pl.ds(0, GCH)], sem_o).wait()
        pltpu.make_async_copy(rows_b, recv_out.at[pl.ds(0, GCH)], sem_o).wait()

    def chunk(i, carry):
        base = wid * ept + i * GCH
        da = do_unit(base, idx_a, rows_a, snd_hbm)
        db = do_unit(base, idx_b, rows_b, rcv_hbm)
        for d_ in da + db:
            d_.wait()
        pltpu.async_copy(rows_a, sent_out.at[pl.ds(base, GCH)], sem_o)
        pltpu.async_copy(rows_b, recv_out.at[pl.ds(base, GCH)], sem_o)
        return carry

    def chunk_drain(i, carry):
        drain_out()
        return chunk(i, carry)

    chunk(0, 0)
    lax.fori_loop(1, nch, chunk_drain, 0)
    drain_out()


_sc_gather = functools.partial(
    pl.kernel,
    out_type=(jax.ShapeDtypeStruct((PE, F), jnp.float32),
              jax.ShapeDtypeStruct((PE, F), jnp.float32)),
    compiler_params=pltpu.CompilerParams(use_tc_tiling_on_sc=False),
    mesh=plsc.VectorSubcoreMesh(core_axis_name="c", subcore_axis_name="s"),
    scratch_types=[
        pltpu.VMEM((GCH,), jnp.int32),
        pltpu.VMEM((GCH,), jnp.int32),
        pltpu.VMEM((GCH, F), jnp.float32),
        pltpu.VMEM((GCH, F), jnp.float32),
        pltpu.SemaphoreType.DMA,
        pltpu.SemaphoreType.DMA,
    ],
)(_sc_gather_body)


def _make_sc_scatter(pd, sch):
    n_chunks = EPT_S // sch
    assert EPT_S % sch == 0 and sch % 128 == 0

    def body(pay_hbm, ridx_hbm, agg_out, pbuf, ridx_v, lidx2, acc):
        c = lax.axis_index("c")
        s = lax.axis_index("s")
        lo = c * R_SC

        def zrow(r, carry):
            for j in range(pd // 16):
                pbuf[r, pl.ds(j * 16, 16)] = jnp.zeros((16,), jnp.float32)
            return carry
        lax.fori_loop(0, sch, zrow, 0)
        zfull, zrem = ZPT // sch, ZPT % sch
        for z in range(zfull):
            pltpu.sync_copy(pbuf, acc.at[pl.ds(s * ZPT + z * sch, sch)])
        if zrem:
            pltpu.sync_copy(pbuf.at[pl.ds(0, zrem)],
                            acc.at[pl.ds(s * ZPT + zfull * sch, zrem)])
        plsc.subcore_barrier()

        def chunk(i, carry):
            base = s * EPT_S + i * sch
            pltpu.sync_copy(ridx_hbm.at[pl.ds(base, sch)], ridx_v)

            def cmp(k, cc):
                v = ridx_v[pl.ds(k * 16, 16)]
                l = v - lo
                ok = (l >= 0) & (l < R_SC)
                l = jnp.where(ok, l, TRASH)
                lidx2[k >> 3, pl.ds((k & 7) * 16, 16)] = l
                return cc
            lax.fori_loop(0, sch // 16, cmp, 0)

            pltpu.sync_copy(pay_hbm.at[pl.ds(base, sch)], pbuf)
            for j in range(sch // 128):
                pltpu.sync_copy(pbuf.at[pl.ds(j * 128, 128)],
                                acc.at[lidx2.at[j]], add=True)
            return carry

        lax.fori_loop(0, n_chunks, chunk, 0)
        plsc.subcore_barrier()
        pltpu.sync_copy(acc.at[pl.ds(s * OPT, OPT)],
                        agg_out.at[pl.ds(lo + s * OPT, OPT)])

    return functools.partial(
        pl.kernel,
        out_type=jax.ShapeDtypeStruct((NAGG, pd), jnp.float32),
        compiler_params=pltpu.CompilerParams(use_tc_tiling_on_sc=False),
        mesh=plsc.VectorSubcoreMesh(core_axis_name="c", subcore_axis_name="s"),
        scratch_types=[
            pltpu.VMEM((sch, pd), jnp.float32),
            pltpu.VMEM((sch,), jnp.int32),
            pltpu.VMEM((sch // 128, 128), jnp.int32),
            pltpu.VMEM_SHARED((ACC_ROWS, pd), jnp.float32),
        ],
    )(body)


_sc_scatter_m = _make_sc_scatter(D, 256)
_sc_scatter_ev = _make_sc_scatter(16, 1024)


def _edge_body(sent_ref, recv_ref, A, B, w20, c0, W1, b1, W2, b2,
               Wx0, bx0, Wx1, bx1, Wx2, bx2, out_m_ref, out_ev_ref):
    sv = sent_ref[...]
    rv = recv_ref[...]
    d = sv - rv
    d3 = d[:, 0:3]
    d2 = jnp.sqrt(jnp.sum(d3 * d3, axis=1, keepdims=True)) ** 2
    d2b = d2.astype(jnp.bfloat16).astype(jnp.float32)
    pre = (_bdot(sv, A[...]) + _bdot(rv, B[...])
           + d2b * w20[...].astype(jnp.float32) + c0[...])
    h = _swish(pre)
    h = _swish(_bdot(h, W1[...]) + b1[...])
    m_ij = _bdot(h, W2[...]) + b2[...]
    t = _swish(_bdot(m_ij, Wx0[...]) + bx0[...])
    t = _swish(_bdot(t, Wx1[...]) + bx1[...])
    xsc = _bdot(t, Wx2[...]) + bx2[...]
    ev = d3 * xsc
    out_m_ref[...] = m_ij
    out_ev_ref[...] = jnp.concatenate(
        [ev, jnp.zeros((m_ij.shape[0], 13), jnp.float32)], axis=-1)


def _tc_edge(sent, recv, ew):
    full = lambda shape: pl.BlockSpec(shape, lambda i: (0, 0))
    in_specs = [
        pl.BlockSpec((BE, F), lambda i: (i, 0)),
        pl.BlockSpec((BE, F), lambda i: (i, 0)),
        full((F, D)), full((F, D)), full((1, D)), full((1, D)),
        full((D, D)), full((1, D)), full((D, D)), full((1, D)),
        full((D, D)), full((1, D)), full((D, D)), full((1, D)),
        full((D, 1)), full((1, 1)),
    ]
    return pl.pallas_call(
        _edge_body,
        grid=(E // BE,),
        in_specs=in_specs,
        out_specs=(pl.BlockSpec((BE, D), lambda i: (i, 0)),
                   pl.BlockSpec((BE, 16), lambda i: (i, 0))),
        out_shape=(jax.ShapeDtypeStruct((PE, D), jnp.float32),
                   jax.ShapeDtypeStruct((PE, 16), jnp.float32)),
        compiler_params=pltpu.CompilerParams(
            dimension_semantics=("arbitrary",)),
    )(sent, recv, *ew)


def _node_body(x_ref, aggm_ref, aggev_ref, Wv0, bv0, Wv1, bv1, Wv2, bv2,
               Wh0a, Wh0b, bh0, Wh1, bh1, Wh2, bh2,
               xs_ref, vs_ref, lns_ref, lnb_ref, out_ref):
    x = x_ref[...]
    m_i = aggm_ref[...]
    sum_x = aggev_ref[...][:, 0:3]
    x_n = x[:, 0:3]
    v_n = x[:, 3:6]
    h_n = x[:, 6:F]

    t = _swish(_bdot(x, Wv0[...]) + bv0[...])
    t = _swish(_bdot(t, Wv1[...]) + bv1[...])
    vfac = _bdot(t, Wv2[...]) + bv2[...]

    v_p = sum_x * (1.0 / (E - 1)) + vfac * v_n
    x_p = x_n + v_p

    u = _swish(_bdot(x, Wh0a[...]) + _bdot(m_i, Wh0b[...]) + bh0[...])
    u = _swish(_bdot(u, Wh1[...]) + bh1[...])
    h_p = _bdot(u, Wh2[...]) + bh2[...] + h_n

    def cnorm(cv, scale):
        nrm = jnp.sqrt(jnp.sum(cv * cv, axis=1, keepdims=True))
        return cv / jnp.maximum(nrm, 1e-5) * scale

    x_p = cnorm(x_p, xs_ref[...])
    v_p = cnorm(v_p, vs_ref[...])

    mu = jnp.mean(h_p, axis=1, keepdims=True)
    var = jnp.mean((h_p - mu) * (h_p - mu), axis=1, keepdims=True)
    h_p = (h_p - mu) * lax.rsqrt(var + 1e-6) * lns_ref[...] + lnb_ref[...]

    out_ref[...] = jnp.concatenate([x_p, v_p, h_p], axis=-1)


def _tc_node(x_all, agg_m, agg_ev, nw):
    full = lambda shape: pl.BlockSpec(shape, lambda i: (0, 0))
    in_specs = [
        pl.BlockSpec((BN, F), lambda i: (i, 0)),
        pl.BlockSpec((BN, D), lambda i: (i, 0)),
        pl.BlockSpec((BN, 16), lambda i: (i, 0)),
        full((F, D)), full((1, D)), full((D, D)), full((1, D)),
        full((D, 1)), full((1, 1)),
        full((F, D)), full((D, D)), full((1, D)),
        full((D, D)), full((1, D)), full((D, H)), full((1, H)),
        full((1, 1)), full((1, 1)), full((1, H)), full((1, H)),
    ]
    return pl.pallas_call(
        _node_body,
        grid=(N // BN,),
        in_specs=in_specs,
        out_specs=pl.BlockSpec((BN, F), lambda i: (i, 0)),
        out_shape=jax.ShapeDtypeStruct((N, F), jnp.float32),
        compiler_params=pltpu.CompilerParams(
            dimension_semantics=("arbitrary",)),
    )(x_all, agg_m, agg_ev, *nw)


def _prep_step_weights(params, g, s):
    p = lambda name: params["s%d_%s" % (s, name)]
    bf = lambda a: a.astype(jnp.bfloat16)
    z = jnp.zeros((F, D), jnp.float32)
    We0 = p("e_W0")
    A = bf(z.at[6:F].set(We0[0:H]))
    B = bf(z.at[6:F].set(We0[H:2 * H]))
    w20 = bf(We0[2 * H:2 * H + 1])
    gb = bf(g).astype(jnp.float32)
    c0 = gb @ bf(We0[2 * H + 1:]).astype(jnp.float32) + p("e_b0")[None, :]
    ew = (A, B, w20, c0,
          bf(p("e_W1")), p("e_b1")[None, :], bf(p("e_W2")), p("e_b2")[None, :],
          bf(p("x_W0")), p("x_b0")[None, :], bf(p("x_W1")), p("x_b1")[None, :],
          bf(p("x_W2")), p("x_b2")[None, :])

    Wv0 = bf(jnp.zeros((F, D), jnp.float32).at[6:F].set(p("v_W0")))
    Wh0 = p("h_W0")
    Wh0a = bf(jnp.zeros((F, D), jnp.float32).at[6:F].set(Wh0[0:H]))
    Wh0b = bf(Wh0[H:])
    nw = (Wv0, p("v_b0")[None, :], bf(p("v_W1")), p("v_b1")[None, :],
          bf(p("v_W2")), p("v_b2")[None, :],
          Wh0a, Wh0b, p("h_b0")[None, :],
          bf(p("h_W1")), p("h_b1")[None, :], bf(p("h_W2")), p("h_b2")[None, :],
          p("xscale")[None, :], p("vscale")[None, :],
          p("ln_scale")[None, :], p("ln_bias")[None, :])
    return ew, nw


def kernel(nodes, senders, receivers, globals_, params):
    g = globals_.reshape(1, -1).astype(jnp.float32)
    senders_pad = jnp.concatenate(
        [senders, jnp.zeros((PE - E,), jnp.int32)])
    receivers_pad0 = jnp.concatenate(
        [receivers, jnp.zeros((PE - E,), jnp.int32)])
    receivers_pad = jnp.concatenate(
        [receivers, jnp.full((PE - E,), N, jnp.int32)])
    x_all = nodes
    for s in range(STEPS):
        ew, nw = _prep_step_weights(params, g, s)
        sent, recv = _sc_gather(x_all, senders_pad, receivers_pad0)
        pay_m, pay_ev = _tc_edge(sent, recv, ew)
        agg_m = _sc_scatter_m(pay_m, receivers_pad)
        agg_ev = _sc_scatter_ev(pay_ev, receivers_pad)
        x_all = _tc_node(x_all, agg_m, agg_ev, nw)
    return x_all
